# Initial kernel scaffold; baseline (speedup 1.0000x reference)
import jax
import jax.numpy as jnp
from jax.experimental import pallas as pl


def kernel(indices, table):
    raise NotImplementedError("write your pallas kernel here")


# speedup vs baseline: 288.7518x; 288.7518x over previous
import jax
import jax.numpy as jnp
from jax.experimental import pallas as pl

VOCAB = 21
EMBED_DIM = 128
PAD_IDX = 20

B_BLK = 256


def _body(idx_ref, tab_ref, out_ref):
    idx = idx_ref[...]
    acc = jnp.zeros((idx.shape[0], EMBED_DIM), jnp.float32)
    inv_l = 1.0 / idx.shape[1]
    for v in range(VOCAB):
        if v == PAD_IDX:
            continue
        cnt = jnp.sum((idx == v).astype(jnp.float32), axis=1, keepdims=True)
        acc = acc + cnt * tab_ref[v, :][None, :]
    out_ref[...] = acc * inv_l


def kernel(indices, table):
    indices = indices.astype(jnp.int32)
    table = table.astype(jnp.float32)
    b, l = indices.shape
    grid = (b // B_BLK,)
    return pl.pallas_call(
        _body,
        grid=grid,
        in_specs=[
            pl.BlockSpec((B_BLK, l), lambda i: (i, 0)),
            pl.BlockSpec((VOCAB, EMBED_DIM), lambda i: (0, 0)),
        ],
        out_specs=pl.BlockSpec((B_BLK, EMBED_DIM), lambda i: (i, 0)),
        out_shape=jax.ShapeDtypeStruct((b, EMBED_DIM), jnp.float32),
    )(indices, table)
